# baseline (speedup 1.0000x reference)
import functools

import jax
import jax.numpy as jnp
from jax import lax
from jax.experimental import pallas as pl
from jax.experimental.pallas import tpu as pltpu
from jax.experimental.pallas import tpu_sc as plsc

_B, _S, _D = 4, 8192, 1024
_TCB = 3
_SEQ_TILE = 2048

_NW = 32
_SEQ_PER_W = _S // _NW
_CHUNK_ROWS = 16
_NCH = _SEQ_PER_W // _CHUNK_ROWS

_mesh = plsc.VectorSubcoreMesh(core_axis_name="c", subcore_axis_name="s")


def _tc_body(x_ref, pos_ref, out_ref):
    out_ref[0] = x_ref[0] + pos_ref[...]


@functools.partial(
    pl.kernel,
    mesh=_mesh,
    out_type=jax.ShapeDtypeStruct((_B - _TCB, _S, _D), jnp.float32),
    scratch_types=[
        pltpu.VMEM((3, _CHUNK_ROWS, _D), jnp.float32),
        pltpu.VMEM((2, _CHUNK_ROWS, _D), jnp.float32),
        pltpu.SemaphoreType.DMA((3,)),
        pltpu.SemaphoreType.DMA((2,)),
        pltpu.SemaphoreType.DMA((3,)),
    ],
)
def _sc_add(x_hbm, pos_hbm, out_hbm, x_v, p_v, xsem, psem, osem):
    wid = lax.axis_index("s") * 2 + lax.axis_index("c")
    seq0 = wid * _SEQ_PER_W

    def x_slice(c):
        return x_hbm.at[_TCB, pl.ds(seq0 + c * _CHUNK_ROWS, _CHUNK_ROWS)]

    def o_slice(c):
        return out_hbm.at[0, pl.ds(seq0 + c * _CHUNK_ROWS, _CHUNK_ROWS)]

    def p_slice(c):
        return pos_hbm.at[pl.ds(seq0 + c * _CHUNK_ROWS, _CHUNK_ROWS)]

    def start_in(c, k):
        pltpu.make_async_copy(x_slice(c), x_v.at[k], xsem.at[k]).start()
        pltpu.make_async_copy(p_slice(c), p_v.at[lax.rem(c, 2)],
                              psem.at[lax.rem(c, 2)]).start()

    def wait_in(c, k):
        pltpu.make_async_copy(x_slice(c), x_v.at[k], xsem.at[k]).wait()
        pltpu.make_async_copy(p_slice(c), p_v.at[lax.rem(c, 2)],
                              psem.at[lax.rem(c, 2)]).wait()

    def start_out(c, k):
        pltpu.make_async_copy(x_v.at[k], o_slice(c), osem.at[k]).start()

    def wait_out(c, k):
        pltpu.make_async_copy(x_v.at[k], o_slice(c), osem.at[k]).wait()

    start_in(0, 0)
    start_in(1, 1)

    def body(g, carry):
        k = lax.rem(g, 3)
        kp = lax.rem(g, 2)
        wait_in(g, k)

        @plsc.parallel_loop(0, _CHUNK_ROWS * _D, 16, unroll=8)
        def _(i):
            r = lax.shift_right_logical(i, 10)
            s = pl.ds(pl.multiple_of(lax.bitwise_and(i, _D - 1), 16), 16)
            x_v[k, r, s] = x_v[k, r, s] + p_v[kp, r, s]

        start_out(g, k)

        @pl.when(g + 2 < _NCH)
        def _():
            @pl.when(g >= 1)
            def _():
                wait_out(g - 1, lax.rem(g + 2, 3))

            start_in(g + 2, lax.rem(g + 2, 3))

        return carry

    lax.fori_loop(0, _NCH, body, 0)
    for c in (_NCH - 3, _NCH - 2, _NCH - 1):
        wait_out(c, c % 3)


def kernel(x, pos_table):
    tc_out = pl.pallas_call(
        _tc_body,
        grid=(_S // _SEQ_TILE, _TCB),
        in_specs=[
            pl.BlockSpec((1, _SEQ_TILE, _D), lambda s, b: (b, s, 0)),
            pl.BlockSpec((_SEQ_TILE, _D), lambda s, b: (s, 0)),
        ],
        out_specs=pl.BlockSpec((1, _SEQ_TILE, _D), lambda s, b: (b, s, 0)),
        out_shape=jax.ShapeDtypeStruct((_TCB, _S, _D), x.dtype),
    )(x, pos_table)
    sc_out = _sc_add(x, pos_table)
    return jnp.concatenate([tc_out, sc_out], axis=0)


# speedup vs baseline: 1.5462x; 1.5462x over previous
import functools

import jax
import jax.numpy as jnp
from jax import lax
from jax.experimental import pallas as pl
from jax.experimental.pallas import tpu as pltpu
from jax.experimental.pallas import tpu_sc as plsc

_B, _S, _D = 4, 8192, 1024
_NW = 32
_SEQ_PER_W = _S // _NW
_CHUNK_ROWS = 16
_P_ROWS = 32
_NPC = _SEQ_PER_W // _P_ROWS
_NCH = (_SEQ_PER_W // _CHUNK_ROWS) * _B

_mesh = plsc.VectorSubcoreMesh(core_axis_name="c", subcore_axis_name="s")


@functools.partial(
    pl.kernel,
    mesh=_mesh,
    out_type=jax.ShapeDtypeStruct((_B, _S, _D), jnp.float32),
    scratch_types=[
        pltpu.VMEM((3, _CHUNK_ROWS, _D), jnp.float32),
        pltpu.VMEM((2, _P_ROWS, _D), jnp.float32),
        pltpu.SemaphoreType.DMA((3,)),
        pltpu.SemaphoreType.DMA((2,)),
        pltpu.SemaphoreType.DMA((3,)),
    ],
)
def _sc_add(x_hbm, pos_hbm, out_hbm, x_v, p_v, xsem, psem, osem):
    wid = lax.axis_index("s") * 2 + lax.axis_index("c")
    seq0 = wid * _SEQ_PER_W

    def row_of(c):
        return seq0 + (c // 8) * _P_ROWS + lax.rem(c, 8) // _B * _CHUNK_ROWS

    def x_slice(c):
        return x_hbm.at[lax.rem(c, _B), pl.ds(row_of(c), _CHUNK_ROWS)]

    def o_slice(c):
        return out_hbm.at[lax.rem(c, _B), pl.ds(row_of(c), _CHUNK_ROWS)]

    def p_slice(pj):
        return pos_hbm.at[pl.ds(seq0 + pj * _P_ROWS, _P_ROWS)]

    def start_in(c, k):
        pltpu.make_async_copy(x_slice(c), x_v.at[k], xsem.at[k]).start()

    def wait_in(c, k):
        pltpu.make_async_copy(x_slice(c), x_v.at[k], xsem.at[k]).wait()

    def start_p(pc, kp):
        pltpu.make_async_copy(p_slice(pc), p_v.at[kp], psem.at[kp]).start()

    def wait_p(pc, kp):
        pltpu.make_async_copy(p_slice(pc), p_v.at[kp], psem.at[kp]).wait()

    def start_out(c, k):
        pltpu.make_async_copy(x_v.at[k], o_slice(c), osem.at[k]).start()

    def wait_out(c, k):
        pltpu.make_async_copy(x_v.at[k], o_slice(c), osem.at[k]).wait()

    start_in(0, 0)
    start_in(1, 1)
    start_p(0, 0)
    start_p(1, 1)

    def body(g, carry):
        k = lax.rem(g, 3)
        t = lax.rem(g, 8)
        pj = g // 8
        kp = lax.rem(pj, 2)
        sub = t // _B

        @pl.when(t == 0)
        def _():
            wait_p(pj, kp)

            @pl.when(jnp.logical_and(pj >= 1, pj + 1 < _NPC))
            def _():
                start_p(pj + 1, lax.rem(pj + 1, 2))

        wait_in(g, k)
        pr0 = sub * _CHUNK_ROWS

        @plsc.parallel_loop(0, _CHUNK_ROWS * _D, 16, unroll=8)
        def _(i):
            r = lax.shift_right_logical(i, 10)
            s = pl.ds(pl.multiple_of(lax.bitwise_and(i, _D - 1), 16), 16)
            x_v[k, r, s] = x_v[k, r, s] + p_v[kp, pr0 + r, s]

        start_out(g, k)

        @pl.when(g + 2 < _NCH)
        def _():
            @pl.when(g >= 1)
            def _():
                wait_out(g - 1, lax.rem(g + 2, 3))

            start_in(g + 2, lax.rem(g + 2, 3))

        return carry

    lax.fori_loop(0, _NCH, body, 0)
    for c in (_NCH - 3, _NCH - 2, _NCH - 1):
        wait_out(c, c % 3)


def kernel(x, pos_table):
    return _sc_add(x, pos_table)


# speedup vs baseline: 1.5734x; 1.0176x over previous
import functools

import jax
import jax.numpy as jnp
from jax import lax
from jax.experimental import pallas as pl
from jax.experimental.pallas import tpu as pltpu
from jax.experimental.pallas import tpu_sc as plsc

_B, _S, _D = 4, 8192, 1024
_NW = 32
_SEQ_PER_W = _S // _NW
_CHUNK_ROWS = 16
_NPC = _SEQ_PER_W // _CHUNK_ROWS
_NCH = _NPC * _B

_mesh = plsc.VectorSubcoreMesh(core_axis_name="c", subcore_axis_name="s")


@functools.partial(
    pl.kernel,
    mesh=_mesh,
    out_type=jax.ShapeDtypeStruct((_B, _S, _D), jnp.float32),
    scratch_types=[
        pltpu.VMEM((3, _CHUNK_ROWS, _D), jnp.float32),
        pltpu.VMEM((2, _CHUNK_ROWS, _D), jnp.float32),
        pltpu.SemaphoreType.DMA((3,)),
        pltpu.SemaphoreType.DMA((2,)),
        pltpu.SemaphoreType.DMA((3,)),
    ],
)
def _sc_add(x_hbm, pos_hbm, out_hbm, x_v, p_v, xsem, psem, osem):
    wid = lax.axis_index("s") * 2 + lax.axis_index("c")
    seq0 = wid * _SEQ_PER_W

    def x_slice(c):
        return x_hbm.at[lax.rem(c, _B),
                        pl.ds(seq0 + (c // _B) * _CHUNK_ROWS, _CHUNK_ROWS)]

    def o_slice(c):
        return out_hbm.at[lax.rem(c, _B),
                          pl.ds(seq0 + (c // _B) * _CHUNK_ROWS, _CHUNK_ROWS)]

    def p_slice(pc):
        return pos_hbm.at[pl.ds(seq0 + pc * _CHUNK_ROWS, _CHUNK_ROWS)]

    def start_in(c, k):
        pltpu.make_async_copy(x_slice(c), x_v.at[k], xsem.at[k]).start()

    def wait_in(c, k):
        pltpu.make_async_copy(x_slice(c), x_v.at[k], xsem.at[k]).wait()

    def start_p(pc, kp):
        pltpu.make_async_copy(p_slice(pc), p_v.at[kp], psem.at[kp]).start()

    def wait_p(pc, kp):
        pltpu.make_async_copy(p_slice(pc), p_v.at[kp], psem.at[kp]).wait()

    def start_out(c, k):
        pltpu.make_async_copy(x_v.at[k], o_slice(c), osem.at[k]).start()

    def wait_out(c, k):
        pltpu.make_async_copy(x_v.at[k], o_slice(c), osem.at[k]).wait()

    start_in(0, 0)
    start_in(1, 1)
    start_p(0, 0)
    start_p(1, 1)

    def body(g, carry):
        k = lax.rem(g, 3)
        b = lax.rem(g, _B)
        pc = g // _B
        kp = lax.rem(pc, 2)

        @pl.when(b == 0)
        def _():
            wait_p(pc, kp)

            @pl.when(jnp.logical_and(pc >= 1, pc + 1 < _NPC))
            def _():
                start_p(pc + 1, lax.rem(pc + 1, 2))

        wait_in(g, k)

        @plsc.parallel_loop(0, _CHUNK_ROWS * _D, 16, unroll=8)
        def _(i):
            r = lax.shift_right_logical(i, 10)
            s = pl.ds(pl.multiple_of(lax.bitwise_and(i, _D - 1), 16), 16)
            x_v[k, r, s] = x_v[k, r, s] + p_v[kp, r, s]

        start_out(g, k)

        @pl.when(g + 2 < _NCH)
        def _():
            @pl.when(g >= 1)
            def _():
                wait_out(g - 1, lax.rem(g + 2, 3))

            start_in(g + 2, lax.rem(g + 2, 3))

        return carry

    lax.fori_loop(0, _NCH, body, 0)
    for c in (_NCH - 3, _NCH - 2, _NCH - 1):
        wait_out(c, c % 3)


def kernel(x, pos_table):
    return _sc_add(x, pos_table)
